# baseline (speedup 1.0000x reference)
import functools

import jax
import jax.numpy as jnp
from jax import lax
from jax.experimental import pallas as pl
from jax.experimental.pallas import tpu as pltpu
from jax.experimental.pallas import tpu_sc as plsc

VOCAB = 64
HID = 8
BATCH = 32
SEQ = 8192
NTOK = BATCH * SEQ
NW = 32
TOK_PER_W = NTOK // NW
CHUNK = 512
NCHUNK = TOK_PER_W // CHUNK
ROWS = CHUNK // 128
IDS_ROWS = NTOK // 128
NGRP = CHUNK // 16

_C23 = 8388608.0


def _sc_body(ids_hbm, w2d_hbm, logits_hbm, hidden_hbm,
             ids0, ids1, hid0, hid1, log0, log1, sav0, sav1,
             w_v, idx_tab, val_tab,
             sem_i0, sem_i1, sem_g0, sem_g1, sem_o0, sem_o1):
    wid = lax.axis_index("s") * 2 + lax.axis_index("c")
    iota = lax.iota(jnp.int32, 16)
    zf = jnp.zeros((16,), jnp.float32)
    ids_b = (ids0, ids1)
    hid_b = (hid0, hid1)
    log_b = (log0, log1)
    sav_b = (sav0, sav1)
    sem_i = (sem_i0, sem_i1)
    sem_g = (sem_g0, sem_g1)
    sem_o = (sem_o0, sem_o1)

    def row0(c):
        return wid * (TOK_PER_W // 128) + c * ROWS

    def tok0(c):
        return wid * TOK_PER_W + c * CHUNK

    pltpu.sync_copy(w2d_hbm, w_v)
    for vg in range(VOCAB // 16):
        v16 = iota + vg * 16
        w0 = plsc.load_gather(w_v, [v16, iota * 0])
        t = (w0 * 10.0 + _C23) - _C23
        t = jnp.maximum(t, 0.0)
        i16 = lax.bitwise_and(t.astype(jnp.int32), VOCAB - 1)
        idx_tab[pl.ds(vg * 16, 16)] = i16
        val_tab[pl.ds(vg * 16, 16)] = i16.astype(jnp.float32) / 10.0

    def zero_body(i, carry):
        for q in range(VOCAB // 16):
            log0[i, pl.ds(q * 16, 16)] = zf
            log1[i, pl.ds(q * 16, 16)] = zf
        return carry

    lax.fori_loop(0, CHUNK, zero_body, None)

    pltpu.async_copy(ids_hbm.at[pl.ds(row0(0), ROWS)], ids0, sem_i0)

    def do_chunk(c, b):
        nb = 1 - b
        pltpu.make_async_copy(
            ids_hbm.at[pl.ds(row0(c), ROWS)], ids_b[b], sem_i[b]).wait()

        @pl.when(c >= 2)
        def _():
            pltpu.make_async_copy(
                log_b[b], logits_hbm.at[pl.ds(tok0(c - 2), CHUNK)],
                sem_o[b]).wait()
            pltpu.make_async_copy(
                hid_b[b], hidden_hbm.at[pl.ds(row0(c - 2), ROWS)],
                sem_o[b]).wait()

        for j in range(ROWS):
            pltpu.async_copy(w2d_hbm.at[ids_b[b].at[j]], hid_b[b].at[j],
                             sem_g[b])

        @pl.when(c + 1 < NCHUNK)
        def _():
            pltpu.async_copy(ids_hbm.at[pl.ds(row0(c + 1), ROWS)],
                             ids_b[nb], sem_i[nb])

        @pl.when(c >= 2)
        def _():
            for g in range(NGRP):
                row16 = iota + g * 16
                i16 = sav_b[b][pl.ds(g * 16, 16)]
                plsc.store_scatter(log_b[b], [row16, i16], zf)

        for g in range(NGRP):
            j, o = divmod(g, 8)
            ids16 = ids_b[b][j, pl.ds(o * 16, 16)]
            i16 = plsc.load_gather(idx_tab, [ids16])
            v16 = plsc.load_gather(val_tab, [ids16])
            row16 = iota + g * 16
            plsc.store_scatter(log_b[b], [row16, i16], v16)
            sav_b[b][pl.ds(g * 16, 16)] = i16

        for j in range(ROWS):
            pltpu.make_async_copy(w2d_hbm.at[ids_b[b].at[j]],
                                  hid_b[b].at[j], sem_g[b]).wait()

        pltpu.async_copy(log_b[b], logits_hbm.at[pl.ds(tok0(c), CHUNK)],
                         sem_o[b])
        pltpu.async_copy(hid_b[b], hidden_hbm.at[pl.ds(row0(c), ROWS)],
                         sem_o[b])

    def loop_body(jj, carry):
        do_chunk(2 * jj, 0)
        do_chunk(2 * jj + 1, 1)
        return carry

    lax.fori_loop(0, NCHUNK // 2, loop_body, None)

    for c in (NCHUNK - 2, NCHUNK - 1):
        b = c % 2
        pltpu.make_async_copy(
            log_b[b], logits_hbm.at[pl.ds(tok0(c), CHUNK)], sem_o[b]).wait()
        pltpu.make_async_copy(
            hid_b[b], hidden_hbm.at[pl.ds(row0(c), ROWS)], sem_o[b]).wait()


@functools.partial(
    pl.kernel,
    out_type=[
        jax.ShapeDtypeStruct((NTOK, VOCAB), jnp.float32),
        jax.ShapeDtypeStruct((IDS_ROWS, 128, HID), jnp.float32),
    ],
    mesh=plsc.VectorSubcoreMesh(core_axis_name="c", subcore_axis_name="s"),
    compiler_params=pltpu.CompilerParams(
        needs_layout_passes=False, use_tc_tiling_on_sc=False),
    scratch_types=[
        pltpu.VMEM((ROWS, 128), jnp.int32),
        pltpu.VMEM((ROWS, 128), jnp.int32),
        pltpu.VMEM((ROWS, 128, HID), jnp.float32),
        pltpu.VMEM((ROWS, 128, HID), jnp.float32),
        pltpu.VMEM((CHUNK, VOCAB), jnp.float32),
        pltpu.VMEM((CHUNK, VOCAB), jnp.float32),
        pltpu.VMEM((CHUNK,), jnp.int32),
        pltpu.VMEM((CHUNK,), jnp.int32),
        pltpu.VMEM((VOCAB, HID), jnp.float32),
        pltpu.VMEM((VOCAB,), jnp.int32),
        pltpu.VMEM((VOCAB,), jnp.float32),
        pltpu.SemaphoreType.DMA,
        pltpu.SemaphoreType.DMA,
        pltpu.SemaphoreType.DMA,
        pltpu.SemaphoreType.DMA,
        pltpu.SemaphoreType.DMA,
        pltpu.SemaphoreType.DMA,
    ],
)
def _fake_model_sc(*refs):
    _sc_body(*refs)


def kernel(input_ids, embedding_weight):
    ids = input_ids.astype(jnp.int32).reshape(IDS_ROWS, 128)
    w = embedding_weight.astype(jnp.float32)
    logits_flat, hidden3 = _fake_model_sc(ids, w)
    return (logits_flat.reshape(BATCH, SEQ, VOCAB),
            hidden3.reshape(BATCH, SEQ, HID))


# speedup vs baseline: 1.7814x; 1.7814x over previous
import functools

import jax
import jax.numpy as jnp
from jax import lax
from jax.experimental import pallas as pl
from jax.experimental.pallas import tpu as pltpu
from jax.experimental.pallas import tpu_sc as plsc

VOCAB = 64
HID = 8
BATCH = 32
SEQ = 8192
NTOK = BATCH * SEQ
NW = 32
TOK_PER_W = NTOK // NW
CHUNK = 512
NCHUNK = TOK_PER_W // CHUNK
ROWS = CHUNK // 128
IDS_ROWS = NTOK // 128
NGRP = CHUNK // 16

_C23 = 8388608.0


def _sc_body(ids_hbm, w2d_hbm, logits_hbm, hidden_hbm,
             ids0, ids1, hid0, hid1, log0, log1, sav0, sav1,
             w_v, idx_tab, val_tab,
             sem_i0, sem_i1, sem_g0, sem_g1, sem_o0, sem_o1):
    wid = lax.axis_index("s") * 2 + lax.axis_index("c")
    iota = lax.iota(jnp.int32, 16)
    zf = jnp.zeros((16,), jnp.float32)
    ids_b = (ids0, ids1)
    hid_b = (hid0, hid1)
    log_b = (log0, log1)
    sav_b = (sav0, sav1)
    sem_i = (sem_i0, sem_i1)
    sem_g = (sem_g0, sem_g1)
    sem_o = (sem_o0, sem_o1)

    def row0(c):
        return wid * (TOK_PER_W // 128) + c * ROWS

    def tok0(c):
        return wid * TOK_PER_W + c * CHUNK

    pltpu.sync_copy(w2d_hbm, w_v)
    for vg in range(VOCAB // 16):
        v16 = iota + vg * 16
        w0 = plsc.load_gather(w_v, [v16, iota * 0])
        t = (w0 * 10.0 + _C23) - _C23
        t = jnp.maximum(t, 0.0)
        i16 = lax.bitwise_and(t.astype(jnp.int32), VOCAB - 1)
        idx_tab[pl.ds(vg * 16, 16)] = i16
        val_tab[pl.ds(vg * 16, 16)] = i16.astype(jnp.float32) / 10.0

    def zero_body(i, carry):
        for q in range(VOCAB // 16):
            log0[i, pl.ds(q * 16, 16)] = zf
            log1[i, pl.ds(q * 16, 16)] = zf
        return carry

    lax.fori_loop(0, CHUNK, zero_body, None)

    pltpu.async_copy(ids_hbm.at[pl.ds(row0(0), ROWS)], ids0, sem_i0)

    def do_chunk(c, b):
        nb = 1 - b
        pltpu.make_async_copy(
            ids_hbm.at[pl.ds(row0(c), ROWS)], ids_b[b], sem_i[b]).wait()

        @pl.when(c >= 2)
        def _():
            pltpu.make_async_copy(
                log_b[b], logits_hbm.at[pl.ds(tok0(c - 2), CHUNK)],
                sem_o[b]).wait()
            pltpu.make_async_copy(
                hid_b[b], hidden_hbm.at[pl.ds(row0(c - 2), ROWS)],
                sem_o[b]).wait()

        @pl.when(c + 1 < NCHUNK)
        def _():
            pltpu.async_copy(ids_hbm.at[pl.ds(row0(c + 1), ROWS)],
                             ids_b[nb], sem_i[nb])

        @pl.when(c >= 2)
        def _():
            for g in range(NGRP):
                row16 = iota + g * 16
                i16 = sav_b[b][pl.ds(g * 16, 16)]
                plsc.store_scatter(log_b[b], [row16, i16], zf)

        lane_hi = jnp.right_shift(iota, 3)
        kpat = lax.bitwise_and(iota, 7)
        for g in range(NGRP):
            j, o = divmod(g, 8)
            jv = iota * 0 + j
            ids16 = ids_b[b][j, pl.ds(o * 16, 16)]
            i16 = plsc.load_gather(idx_tab, [ids16])
            v16 = plsc.load_gather(val_tab, [ids16])
            row16 = iota + g * 16
            plsc.store_scatter(log_b[b], [row16, i16], v16)
            sav_b[b][pl.ds(g * 16, 16)] = i16
            for q in range(8):
                colv = o * 16 + 2 * q + lane_hi
                idv = plsc.load_gather(ids_b[b], [jv, colv])
                hv = plsc.load_gather(w_v, [idv, kpat])
                plsc.store_scatter(hid_b[b], [jv, colv, kpat], hv)

        pltpu.async_copy(log_b[b], logits_hbm.at[pl.ds(tok0(c), CHUNK)],
                         sem_o[b])
        pltpu.async_copy(hid_b[b], hidden_hbm.at[pl.ds(row0(c), ROWS)],
                         sem_o[b])

    def loop_body(jj, carry):
        do_chunk(2 * jj, 0)
        do_chunk(2 * jj + 1, 1)
        return carry

    lax.fori_loop(0, NCHUNK // 2, loop_body, None)

    for c in (NCHUNK - 2, NCHUNK - 1):
        b = c % 2
        pltpu.make_async_copy(
            log_b[b], logits_hbm.at[pl.ds(tok0(c), CHUNK)], sem_o[b]).wait()
        pltpu.make_async_copy(
            hid_b[b], hidden_hbm.at[pl.ds(row0(c), ROWS)], sem_o[b]).wait()


@functools.partial(
    pl.kernel,
    out_type=[
        jax.ShapeDtypeStruct((NTOK, VOCAB), jnp.float32),
        jax.ShapeDtypeStruct((IDS_ROWS, 128, HID), jnp.float32),
    ],
    mesh=plsc.VectorSubcoreMesh(core_axis_name="c", subcore_axis_name="s"),
    compiler_params=pltpu.CompilerParams(
        needs_layout_passes=False, use_tc_tiling_on_sc=False),
    scratch_types=[
        pltpu.VMEM((ROWS, 128), jnp.int32),
        pltpu.VMEM((ROWS, 128), jnp.int32),
        pltpu.VMEM((ROWS, 128, HID), jnp.float32),
        pltpu.VMEM((ROWS, 128, HID), jnp.float32),
        pltpu.VMEM((CHUNK, VOCAB), jnp.float32),
        pltpu.VMEM((CHUNK, VOCAB), jnp.float32),
        pltpu.VMEM((CHUNK,), jnp.int32),
        pltpu.VMEM((CHUNK,), jnp.int32),
        pltpu.VMEM((VOCAB, HID), jnp.float32),
        pltpu.VMEM((VOCAB,), jnp.int32),
        pltpu.VMEM((VOCAB,), jnp.float32),
        pltpu.SemaphoreType.DMA,
        pltpu.SemaphoreType.DMA,
        pltpu.SemaphoreType.DMA,
        pltpu.SemaphoreType.DMA,
        pltpu.SemaphoreType.DMA,
        pltpu.SemaphoreType.DMA,
    ],
)
def _fake_model_sc(*refs):
    _sc_body(*refs)


def kernel(input_ids, embedding_weight):
    ids = input_ids.astype(jnp.int32).reshape(IDS_ROWS, 128)
    w = embedding_weight.astype(jnp.float32)
    logits_flat, hidden3 = _fake_model_sc(ids, w)
    return (logits_flat.reshape(BATCH, SEQ, VOCAB),
            hidden3.reshape(BATCH, SEQ, HID))


# speedup vs baseline: 1.8368x; 1.0311x over previous
import functools

import jax
import jax.numpy as jnp
from jax import lax
from jax.experimental import pallas as pl
from jax.experimental.pallas import tpu as pltpu
from jax.experimental.pallas import tpu_sc as plsc

VOCAB = 64
HID = 8
BATCH = 32
SEQ = 8192
NTOK = BATCH * SEQ
NW = 32
TOK_PER_W = NTOK // NW
CHUNK = 512
NCHUNK = TOK_PER_W // CHUNK
NGRP = CHUNK // 16

_C23 = 8388608.0


def _sc_body(ids_hbm, w_hbm, logits_hbm, hidden_hbm,
             ids0, ids1, hid0, hid1, log0, log1, sav0, sav1,
             tmp_v, w_v, idx_tab, val_tab,
             sem_i0, sem_i1, sem_o0, sem_o1):
    wid = lax.axis_index("s") * 2 + lax.axis_index("c")
    iota = lax.iota(jnp.int32, 16)
    zf = jnp.zeros((16,), jnp.float32)
    iota64 = iota * VOCAB
    lane_hi = jnp.right_shift(iota, 3)
    kpat = lax.bitwise_and(iota, 7)
    ids_b = (ids0, ids1)
    hid_b = (hid0, hid1)
    log_b = (log0, log1)
    sav_b = (sav0, sav1)
    sem_i = (sem_i0, sem_i1)
    sem_o = (sem_o0, sem_o1)

    def tok0(c):
        return wid * TOK_PER_W + c * CHUNK

    pltpu.sync_copy(w_hbm, w_v)
    for vg in range(VOCAB // 16):
        v16 = iota + vg * 16
        w0 = plsc.load_gather(w_v, [v16 * HID])
        t = (w0 * 10.0 + _C23) - _C23
        t = jnp.maximum(t, 0.0)
        i16 = lax.bitwise_and(t.astype(jnp.int32), VOCAB - 1)
        idx_tab[pl.ds(vg * 16, 16)] = i16
        val_tab[pl.ds(vg * 16, 16)] = i16.astype(jnp.float32) / 10.0

    def zero_body(i, carry):
        log0[pl.ds(i * 16, 16)] = zf
        log1[pl.ds(i * 16, 16)] = zf
        return carry

    lax.fori_loop(0, CHUNK * VOCAB // 16, zero_body, None)

    pltpu.async_copy(ids_hbm.at[pl.ds(tok0(0), CHUNK)], ids0, sem_i0)

    def do_chunk(c, b):
        nb = 1 - b
        pltpu.make_async_copy(
            ids_hbm.at[pl.ds(tok0(c), CHUNK)], ids_b[b], sem_i[b]).wait()

        @pl.when(c >= 2)
        def _():
            pltpu.make_async_copy(
                log_b[b],
                logits_hbm.at[pl.ds(tok0(c - 2) * VOCAB, CHUNK * VOCAB)],
                sem_o[b]).wait()
            pltpu.make_async_copy(
                hid_b[b],
                hidden_hbm.at[pl.ds(tok0(c - 2) * HID, CHUNK * HID)],
                sem_o[b]).wait()

        @pl.when(c + 1 < NCHUNK)
        def _():
            pltpu.async_copy(ids_hbm.at[pl.ds(tok0(c + 1), CHUNK)],
                             ids_b[nb], sem_i[nb])

        @pl.when(c >= 2)
        def _():
            for g in range(NGRP):
                p16 = sav_b[b][pl.ds(g * 16, 16)]
                plsc.store_scatter(log_b[b], [p16], zf)

        for g in range(NGRP):
            ids16 = ids_b[b][pl.ds(g * 16, 16)]
            i16 = plsc.load_gather(idx_tab, [ids16])
            v16 = plsc.load_gather(val_tab, [ids16])
            pos16 = iota64 + g * (16 * VOCAB) + i16
            plsc.store_scatter(log_b[b], [pos16], v16)
            sav_b[b][pl.ds(g * 16, 16)] = pos16
            tmp_v[...] = ids16 * HID
            for q in range(8):
                idv = plsc.load_gather(tmp_v, [lane_hi + 2 * q])
                hv = plsc.load_gather(w_v, [idv + kpat])
                hid_b[b][pl.ds(g * 128 + q * 16, 16)] = hv

        pltpu.async_copy(
            log_b[b], logits_hbm.at[pl.ds(tok0(c) * VOCAB, CHUNK * VOCAB)],
            sem_o[b])
        pltpu.async_copy(
            hid_b[b], hidden_hbm.at[pl.ds(tok0(c) * HID, CHUNK * HID)],
            sem_o[b])

    def loop_body(jj, carry):
        do_chunk(2 * jj, 0)
        do_chunk(2 * jj + 1, 1)
        return carry

    lax.fori_loop(0, NCHUNK // 2, loop_body, None)

    for c in (NCHUNK - 2, NCHUNK - 1):
        b = c % 2
        pltpu.make_async_copy(
            log_b[b], logits_hbm.at[pl.ds(tok0(c) * VOCAB, CHUNK * VOCAB)],
            sem_o[b]).wait()
        pltpu.make_async_copy(
            hid_b[b], hidden_hbm.at[pl.ds(tok0(c) * HID, CHUNK * HID)],
            sem_o[b]).wait()


@functools.partial(
    pl.kernel,
    out_type=[
        jax.ShapeDtypeStruct((NTOK * VOCAB,), jnp.float32),
        jax.ShapeDtypeStruct((NTOK * HID,), jnp.float32),
    ],
    mesh=plsc.VectorSubcoreMesh(core_axis_name="c", subcore_axis_name="s"),
    compiler_params=pltpu.CompilerParams(needs_layout_passes=False),
    scratch_types=[
        pltpu.VMEM((CHUNK,), jnp.int32),
        pltpu.VMEM((CHUNK,), jnp.int32),
        pltpu.VMEM((CHUNK * HID,), jnp.float32),
        pltpu.VMEM((CHUNK * HID,), jnp.float32),
        pltpu.VMEM((CHUNK * VOCAB,), jnp.float32),
        pltpu.VMEM((CHUNK * VOCAB,), jnp.float32),
        pltpu.VMEM((CHUNK,), jnp.int32),
        pltpu.VMEM((CHUNK,), jnp.int32),
        pltpu.VMEM((16,), jnp.int32),
        pltpu.VMEM((VOCAB * HID,), jnp.float32),
        pltpu.VMEM((VOCAB,), jnp.int32),
        pltpu.VMEM((VOCAB,), jnp.float32),
        pltpu.SemaphoreType.DMA,
        pltpu.SemaphoreType.DMA,
        pltpu.SemaphoreType.DMA,
        pltpu.SemaphoreType.DMA,
    ],
)
def _fake_model_sc(*refs):
    _sc_body(*refs)


def kernel(input_ids, embedding_weight):
    ids = input_ids.astype(jnp.int32).reshape(NTOK)
    w = embedding_weight.astype(jnp.float32).reshape(VOCAB * HID)
    logits_flat, hidden_flat = _fake_model_sc(ids, w)
    return (logits_flat.reshape(BATCH, SEQ, VOCAB),
            hidden_flat.reshape(BATCH, SEQ, HID))
